# Initial kernel scaffold; baseline (speedup 1.0000x reference)
import jax
import jax.numpy as jnp
from jax.experimental import pallas as pl


def kernel(X, edge_index, edge_weight, H, C, params):
    raise NotImplementedError("write your pallas kernel here")


# speedup vs baseline: 5.2044x; 5.2044x over previous
import functools

import jax
import jax.numpy as jnp
from jax import lax
from jax.experimental import pallas as pl
from jax.experimental.pallas import tpu as pltpu, tpu_sc as plsc

D = 128
LANES = 16
NSUB = 16
NCORES = 2


def _sc_aggregate_body(nrows_pad, n_edge_rows,
                       xp, hp, src2d, dst2d, ew2d,
                       sx, sh,
                       sidx, didx, ewv, rows, cvm, stg, red, sem,
                       acc, shcnt):
    c = lax.axis_index("c")
    s = lax.axis_index("s")
    per_worker = nrows_pad // NSUB
    n_chunks = per_worker // 128
    base = s * per_worker

    def zero_hist(r, _):
        cvm[pl.ds(r * LANES, LANES)] = jnp.zeros((LANES,), jnp.float32)
        return 0
    lax.fori_loop(0, nrows_pad // LANES, zero_hist, 0)

    def zero_rows(r, _):
        for i in range(D // LANES):
            rows[r, pl.ds(i * LANES, LANES)] = jnp.zeros((LANES,), jnp.float32)
        return 0
    lax.fori_loop(0, 128, zero_rows, 0)

    for k in range(n_chunks):
        pltpu.sync_copy(rows, acc.at[pl.ds(base + k * 128, 128)])
    plsc.subcore_barrier()

    per = n_edge_rows // NSUB
    rem = n_edge_rows % NSUB
    my_rows = per + jnp.where(s < rem, 1, 0)
    start = s * per + jnp.minimum(s, rem)

    def make_body(table):
        def body(i, _):
            r = start + i
            pltpu.sync_copy(src2d.at[pl.ds(r, 1)], sidx)
            pltpu.sync_copy(dst2d.at[pl.ds(r, 1)], didx)
            pltpu.sync_copy(ew2d.at[pl.ds(r, 1)], ewv)
            pltpu.async_copy(table.at[sidx.at[0]], rows, sem).wait()

            def scale(g, _):
                wv = ewv[0, pl.ds(g * LANES, LANES)]
                dv = didx[0, pl.ds(g * LANES, LANES)]
                plsc.addupdate_scatter(cvm, [dv], jnp.ones((LANES,), jnp.float32))
                for j in range(LANES):
                    e = g * LANES + j
                    w = wv[j]
                    for i2 in range(D // LANES):
                        sl = pl.ds(i2 * LANES, LANES)
                        rows[e, sl] = rows[e, sl] * w
                return 0
            lax.fori_loop(0, 128 // LANES, scale, 0)

            pltpu.sync_copy(rows, acc.at[didx.at[0]], add=True)
            return 0
        return body

    @pl.when(c == 0)
    def _():
        lax.fori_loop(0, my_rows, make_body(xp), 0)

    @pl.when(c == 1)
    def _():
        lax.fori_loop(0, my_rows, make_body(hp), 0)

    pltpu.sync_copy(cvm, shcnt.at[s])
    plsc.subcore_barrier()
    pltpu.sync_copy(shcnt.at[:, pl.ds(base, per_worker)], stg)

    def reduce_cnt(r, _):
        tot = jnp.zeros((LANES,), jnp.float32)
        for w in range(NSUB):
            tot = tot + stg[w, pl.ds(r * LANES, LANES)]
        red[pl.ds(r * LANES, LANES)] = 1.0 / jnp.maximum(tot, 1.0)
        return 0
    lax.fori_loop(0, per_worker // LANES, reduce_cnt, 0)

    def make_writeout(out):
        def wo(k, _):
            pltpu.sync_copy(acc.at[pl.ds(base + k * 128, 128)], rows)

            def norm(g, _):
                iv = red[pl.ds(k * 128 + g * LANES, LANES)]
                for j in range(LANES):
                    e = g * LANES + j
                    w = iv[j]
                    for i2 in range(D // LANES):
                        sl = pl.ds(i2 * LANES, LANES)
                        rows[e, sl] = rows[e, sl] * w
                return 0
            lax.fori_loop(0, 128 // LANES, norm, 0)
            pltpu.sync_copy(rows, out.at[pl.ds(base + k * 128, 128)])
            return 0
        lax.fori_loop(0, n_chunks, wo, 0)

    @pl.when(c == 0)
    def _():
        make_writeout(sx)

    @pl.when(c == 1)
    def _():
        make_writeout(sh)


@functools.partial(jax.jit, static_argnums=(0, 1))
def _sc_aggregate(nrows_pad, n_edge_rows, xp, hp, src2d, dst2d, ew2d):
    mesh = plsc.VectorSubcoreMesh(core_axis_name="c", subcore_axis_name="s",
                                  num_cores=NCORES, num_subcores=NSUB)
    f = pl.kernel(
        functools.partial(_sc_aggregate_body, nrows_pad, n_edge_rows),
        out_type=[
            jax.ShapeDtypeStruct((nrows_pad, D), jnp.float32),
            jax.ShapeDtypeStruct((nrows_pad, D), jnp.float32),
        ],
        mesh=mesh,
        compiler_params=pltpu.CompilerParams(needs_layout_passes=False),
        scratch_types=[
            pltpu.VMEM((1, 128), jnp.int32),
            pltpu.VMEM((1, 128), jnp.int32),
            pltpu.VMEM((1, 128), jnp.float32),
            pltpu.VMEM((128, D), jnp.float32),
            pltpu.VMEM((nrows_pad,), jnp.float32),
            pltpu.VMEM((NSUB, nrows_pad // NSUB), jnp.float32),
            pltpu.VMEM((nrows_pad // NSUB,), jnp.float32),
            pltpu.SemaphoreType.DMA,
            pltpu.VMEM_SHARED((nrows_pad, D), jnp.float32),
            pltpu.VMEM_SHARED((NSUB, nrows_pad), jnp.float32),
        ],
    )
    return f(xp, hp, src2d, dst2d, ew2d)


def _tc_lstm_body(ax_ref, x, ah_ref, h, cb, w1, w2, w3, w4, bias,
                  wci, wcf, wco, h2, c2):
    p = (jnp.dot(ax_ref[...], w1[...], preferred_element_type=jnp.float32)
         + jnp.dot(x[...], w2[...], preferred_element_type=jnp.float32)
         + jnp.dot(ah_ref[...], w3[...], preferred_element_type=jnp.float32)
         + jnp.dot(h[...], w4[...], preferred_element_type=jnp.float32)
         + bias[...])
    cc = cb[...]
    gi = jax.nn.sigmoid(p[:, 0:D] + wci[...] * cc)
    gf = jax.nn.sigmoid(p[:, D:2 * D] + wcf[...] * cc)
    gt = jnp.tanh(p[:, 2 * D:3 * D])
    c_new = gf * cc + gi * gt
    go = jax.nn.sigmoid(p[:, 3 * D:4 * D] + wco[...] * c_new)
    h2[...] = go * jnp.tanh(c_new)
    c2[...] = c_new


@functools.partial(jax.jit, static_argnums=(0,))
def _tc_lstm(nrows_pad, ax, xp, ah, hp, cp, w1, w2, w3, w4, bias,
             wci, wcf, wco):
    bm = 512
    grid = (nrows_pad // bm,)
    row_spec = pl.BlockSpec((bm, D), lambda i: (i, 0))
    w_spec = pl.BlockSpec((D, 4 * D), lambda i: (0, 0))
    b_spec = pl.BlockSpec((1, 4 * D), lambda i: (0, 0))
    v_spec = pl.BlockSpec((1, D), lambda i: (0, 0))
    return pl.pallas_call(
        _tc_lstm_body,
        grid=grid,
        in_specs=[row_spec, row_spec, row_spec, row_spec, row_spec,
                  w_spec, w_spec, w_spec, w_spec, b_spec,
                  v_spec, v_spec, v_spec],
        out_specs=[row_spec, row_spec],
        out_shape=[
            jax.ShapeDtypeStruct((nrows_pad, D), jnp.float32),
            jax.ShapeDtypeStruct((nrows_pad, D), jnp.float32),
        ],
    )(ax, xp, ah, hp, cp, w1, w2, w3, w4, bias, wci, wcf, wco)


def kernel(X, edge_index, edge_weight, H, C, params):
    n = X.shape[0]
    e = edge_weight.shape[0]
    assert e % 128 == 0
    n_edge_rows = e // 128
    nrows_pad = -(-n // 2048) * 2048

    pad = [(0, nrows_pad - n), (0, 0)]
    xp = jnp.pad(X, pad)
    hp = jnp.pad(H, pad)
    cp = jnp.pad(C, pad)
    src2d = edge_index[0].reshape(n_edge_rows, 128)
    dst2d = edge_index[1].reshape(n_edge_rows, 128)
    ew2d = edge_weight.reshape(n_edge_rows, 128)

    p = params
    gates = ['i', 'f', 'c', 'o']
    w1 = jnp.concatenate([p['W_l_x_' + g] for g in gates], axis=1)
    w2 = jnp.concatenate([p['W_r_x_' + g] for g in gates], axis=1)
    w3 = jnp.concatenate([p['W_l_h_' + g] for g in gates], axis=1)
    w4 = jnp.concatenate([p['W_r_h_' + g] for g in gates], axis=1)
    bias = jnp.concatenate(
        [p['bc_x_' + g] + p['bc_h_' + g] + p['b_' + g][0] for g in gates]
    ).reshape(1, 4 * D)

    ax, ah = _sc_aggregate(nrows_pad, n_edge_rows,
                           xp, hp, src2d, dst2d, ew2d)
    h2, c2 = _tc_lstm(nrows_pad, ax, xp, ah, hp, cp,
                      w1, w2, w3, w4, bias,
                      p['w_c_i'], p['w_c_f'], p['w_c_o'])
    return h2[:n], c2[:n]
